# baseline (speedup 1.0000x reference)
import functools

import jax
import jax.numpy as jnp
from jax import lax
from jax.experimental import pallas as pl
from jax.experimental.pallas import tpu as pltpu
from jax.experimental.pallas import tpu_sc as plsc

H = 32
NC = 2
NS = 16
NW = NC * NS
CHUNK = 128
W1COL = 40
BN = 2000


def _tc_proj1(x, w1, bl1, n):
    def body(x_ref, w_ref, b_ref, yt_ref, r_ref):
        z = lax.dot_general(x_ref[...], w_ref[...], (((1,), (1,)), ((), ())),
                            preferred_element_type=jnp.float32)
        yt_ref[...] = jnp.concatenate(
            [z[:, :H], jnp.ones((BN, 1), jnp.float32),
             jnp.zeros((BN, W1COL - H - 1), jnp.float32)], axis=1)
        r_ref[...] = z[:, H:] + b_ref[...]

    d = x.shape[1]
    return pl.pallas_call(
        body,
        grid=(n // BN,),
        in_specs=[pl.BlockSpec((BN, d), lambda i: (i, 0)),
                  pl.BlockSpec((2 * H, d), lambda i: (0, 0)),
                  pl.BlockSpec((1, H), lambda i: (0, 0))],
        out_specs=[pl.BlockSpec((BN, W1COL), lambda i: (i, 0)),
                   pl.BlockSpec((BN, H), lambda i: (i, 0))],
        out_shape=[jax.ShapeDtypeStruct((n, W1COL), jnp.float32),
                   jax.ShapeDtypeStruct((n, H), jnp.float32)],
    )(x, w1, bl1.reshape(1, H))


def _tc_mid(acc1, r1, w2, bl2, n, np_rows):
    def body(a_ref, r1_ref, w_ref, b_ref, y2_ref, r2_ref):
        s = a_ref[0] + a_ref[1]
        c = jnp.maximum(s[:, H:H + 1], 1.0)
        h1 = jnp.maximum(s[:, :H] / c + r1_ref[...], 0.0)
        z = lax.dot_general(h1, w_ref[...], (((1,), (1,)), ((), ())),
                            preferred_element_type=jnp.float32)
        y2_ref[...] = z[:, :H]
        r2_ref[...] = z[:, H:] + b_ref[...]

    return pl.pallas_call(
        body,
        grid=(n // BN,),
        in_specs=[pl.BlockSpec((NC, BN, W1COL), lambda i: (0, i, 0)),
                  pl.BlockSpec((BN, H), lambda i: (i, 0)),
                  pl.BlockSpec((2 * H, H), lambda i: (0, 0)),
                  pl.BlockSpec((1, H), lambda i: (0, 0))],
        out_specs=[pl.BlockSpec((BN, H), lambda i: (i, 0)),
                   pl.BlockSpec((BN, H), lambda i: (i, 0))],
        out_shape=[jax.ShapeDtypeStruct((n, H), jnp.float32),
                   jax.ShapeDtypeStruct((n, H), jnp.float32)],
    )(acc1, r1, w2, bl2.reshape(1, H))


def _tc_head(acc2, acc1, r2, wcat, bcat, n, nout):
    def body(a2_ref, a1_ref, r2_ref, w_ref, b_ref, o_ref):
        s2 = a2_ref[0] + a2_ref[1]
        cs = a1_ref[0] + a1_ref[1]
        c = jnp.maximum(cs[:, H:H + 1], 1.0)
        h2 = jnp.maximum(s2 / c + r2_ref[...], 0.0)
        o_ref[...] = lax.dot_general(h2, w_ref[...], (((1,), (1,)), ((), ())),
                                     preferred_element_type=jnp.float32) + b_ref[...]

    return pl.pallas_call(
        body,
        grid=(n // BN,),
        in_specs=[pl.BlockSpec((NC, BN, H), lambda i: (0, i, 0)),
                  pl.BlockSpec((NC, BN, W1COL), lambda i: (0, i, 0)),
                  pl.BlockSpec((BN, H), lambda i: (i, 0)),
                  pl.BlockSpec((nout, H), lambda i: (0, 0)),
                  pl.BlockSpec((1, nout), lambda i: (0, 0))],
        out_specs=pl.BlockSpec((BN, nout), lambda i: (i, 0)),
        out_shape=jax.ShapeDtypeStruct((n, nout), jnp.float32),
    )(acc2, acc1, r2, wcat, bcat.reshape(1, nout))


def _make_edge_pass(width, nch, np_rows):
    rows_pt = np_rows // NS
    mesh = plsc.VectorSubcoreMesh(core_axis_name="c", subcore_axis_name="s")

    @functools.partial(
        pl.kernel, mesh=mesh,
        compiler_params=pltpu.CompilerParams(use_tc_tiling_on_sc=False),
        out_type=jax.ShapeDtypeStruct((NC, np_rows, width), jnp.float32),
        scratch_types=[
            pltpu.VMEM((nch, CHUNK), jnp.int32),
            pltpu.VMEM((nch, CHUNK), jnp.int32),
            pltpu.VMEM((CHUNK, width), jnp.float32),
            pltpu.VMEM((CHUNK, width), jnp.float32),
            pltpu.VMEM((16, width), jnp.float32),
            pltpu.VMEM_SHARED((np_rows, width), jnp.float32),
            pltpu.SemaphoreType.DMA,
            pltpu.SemaphoreType.DMA,
        ],
    )
    def edge_pass(ytab, src_idx, dst_idx, out_acc,
                  src_v, dst_v, rows_a, rows_b, zbuf, acc_sh, sem_a, sem_b):
        cid = lax.axis_index("c")
        sid = lax.axis_index("s")
        wid = sid * NC + cid

        for co in (0, 16, width - 16):
            for r in range(16):
                zbuf[r, pl.ds(co, 16)] = jnp.zeros((16,), jnp.float32)
        row0 = sid * rows_pt

        def zloop(k, carry):
            pltpu.sync_copy(zbuf, acc_sh.at[pl.ds(row0 + k * 16, 16)])
            return carry
        lax.fori_loop(0, rows_pt // 16, zloop, 0)

        pltpu.sync_copy(src_idx.at[wid], src_v)
        pltpu.sync_copy(dst_idx.at[wid], dst_v)
        plsc.subcore_barrier()

        pltpu.async_copy(ytab.at[src_v.at[0]], rows_a, sem_a)

        def body(i, carry):
            j0 = 2 * i
            j1 = j0 + 1
            pltpu.async_copy(ytab.at[src_v.at[j1]], rows_b, sem_b)
            pltpu.make_async_copy(ytab.at[src_v.at[j0]], rows_a, sem_a).wait()
            pltpu.sync_copy(rows_a, acc_sh.at[dst_v.at[j0]], add=True)
            j2 = jnp.minimum(j0 + 2, nch - 1)
            pltpu.async_copy(ytab.at[src_v.at[j2]], rows_a, sem_a)
            pltpu.make_async_copy(ytab.at[src_v.at[j1]], rows_b, sem_b).wait()
            pltpu.sync_copy(rows_b, acc_sh.at[dst_v.at[j1]], add=True)
            return carry
        lax.fori_loop(0, nch // 2, body, 0)
        pltpu.make_async_copy(ytab.at[src_v.at[0]], rows_a, sem_a).wait()

        plsc.subcore_barrier()
        pltpu.sync_copy(acc_sh.at[pl.ds(row0, rows_pt)],
                        out_acc.at[cid, pl.ds(row0, rows_pt)])

    return edge_pass


def kernel(x, edge_index, Wl1, bl1, Wr1, Wl2, bl2, Wr2, Wh, bh, Wm, bm):
    n, d = x.shape
    e = edge_index.shape[1]
    nout = Wh.shape[0] + Wm.shape[0]

    epw = -(-e // (NW * 2 * CHUNK)) * 2 * CHUNK
    e_pad = NW * epw
    nch = epw // CHUNK
    np_rows = -(-(n + 1) // (NS * 16)) * NS * 16
    src = jnp.concatenate([edge_index[0], jnp.zeros((e_pad - e,), jnp.int32)])
    dst = jnp.concatenate([edge_index[1], jnp.full((e_pad - e,), n, jnp.int32)])
    src_r = src.reshape(NW, nch, CHUNK)
    dst_r = dst.reshape(NW, nch, CHUNK)

    w1 = jnp.concatenate([Wl1, Wr1], axis=0)
    w2 = jnp.concatenate([Wl2, Wr2], axis=0)
    wcat = jnp.concatenate([Wh, Wm], axis=0)
    bcat = jnp.concatenate([bh, bm])

    ytab, r1 = _tc_proj1(x, w1, bl1, n)
    acc1 = _make_edge_pass(W1COL, nch, np_rows)(ytab, src_r, dst_r)
    y2, r2 = _tc_mid(acc1, r1, w2, bl2, n, np_rows)
    acc2 = _make_edge_pass(H, nch, np_rows)(y2, src_r, dst_r)
    out = _tc_head(acc2, acc1, r2, wcat, bcat, n, nout)
    return out[:, :Wh.shape[0]], out[:, Wh.shape[0]:]


# speedup vs baseline: 1.6846x; 1.6846x over previous
import functools

import jax
import jax.numpy as jnp
from jax import lax
from jax.experimental import pallas as pl
from jax.experimental.pallas import tpu as pltpu
from jax.experimental.pallas import tpu_sc as plsc

H = 32
NC = 2
NS = 16
NW = NC * NS
CHUNK = 128
W1COL = 40
BN = 1280


def _tc_proj1(x, w1, bl1, n):
    def body(x_ref, w_ref, b_ref, yt_ref, r_ref):
        z = lax.dot_general(x_ref[...], w_ref[...], (((1,), (1,)), ((), ())),
                            preferred_element_type=jnp.float32)
        yt_ref[...] = jnp.concatenate(
            [z[:, :H], jnp.ones((BN, 1), jnp.float32),
             jnp.zeros((BN, W1COL - H - 1), jnp.float32)], axis=1)
        r_ref[...] = z[:, H:] + b_ref[...]

    d = x.shape[1]
    return pl.pallas_call(
        body,
        grid=(n // BN,),
        in_specs=[pl.BlockSpec((BN, d), lambda i: (i, 0)),
                  pl.BlockSpec((2 * H, d), lambda i: (0, 0)),
                  pl.BlockSpec((1, H), lambda i: (0, 0))],
        out_specs=[pl.BlockSpec((BN, W1COL), lambda i: (i, 0)),
                   pl.BlockSpec((BN, H), lambda i: (i, 0))],
        out_shape=[jax.ShapeDtypeStruct((n, W1COL), jnp.float32),
                   jax.ShapeDtypeStruct((n, H), jnp.float32)],
    )(x, w1, bl1.reshape(1, H))


def _tc_mid(acc1, r1, w2, bl2, n, np_rows):
    def body(a_ref, r1_ref, w_ref, b_ref, y2_ref, r2_ref):
        s = a_ref[0] + a_ref[1]
        c = jnp.maximum(s[:, H:H + 1], 1.0)
        h1 = jnp.maximum(s[:, :H] / c + r1_ref[...], 0.0)
        z = lax.dot_general(h1, w_ref[...], (((1,), (1,)), ((), ())),
                            preferred_element_type=jnp.float32)
        y2_ref[...] = z[:, :H]
        r2_ref[...] = z[:, H:] + b_ref[...]

    return pl.pallas_call(
        body,
        grid=(n // BN,),
        in_specs=[pl.BlockSpec((NC, BN, W1COL), lambda i: (0, i, 0)),
                  pl.BlockSpec((BN, H), lambda i: (i, 0)),
                  pl.BlockSpec((2 * H, H), lambda i: (0, 0)),
                  pl.BlockSpec((1, H), lambda i: (0, 0))],
        out_specs=[pl.BlockSpec((BN, H), lambda i: (i, 0)),
                   pl.BlockSpec((BN, H), lambda i: (i, 0))],
        out_shape=[jax.ShapeDtypeStruct((n, H), jnp.float32),
                   jax.ShapeDtypeStruct((n, H), jnp.float32)],
    )(acc1, r1, w2, bl2.reshape(1, H))


def _tc_head(acc2, acc1, r2, wcat, bcat, n, nout):
    def body(a2_ref, a1_ref, r2_ref, w_ref, b_ref, o_ref):
        s2 = a2_ref[0] + a2_ref[1]
        cs = a1_ref[0] + a1_ref[1]
        c = jnp.maximum(cs[:, H:H + 1], 1.0)
        h2 = jnp.maximum(s2 / c + r2_ref[...], 0.0)
        o_ref[...] = lax.dot_general(h2, w_ref[...], (((1,), (1,)), ((), ())),
                                     preferred_element_type=jnp.float32) + b_ref[...]

    return pl.pallas_call(
        body,
        grid=(n // BN,),
        in_specs=[pl.BlockSpec((NC, BN, H), lambda i: (0, i, 0)),
                  pl.BlockSpec((NC, BN, W1COL), lambda i: (0, i, 0)),
                  pl.BlockSpec((BN, H), lambda i: (i, 0)),
                  pl.BlockSpec((nout, H), lambda i: (0, 0)),
                  pl.BlockSpec((1, nout), lambda i: (0, 0))],
        out_specs=pl.BlockSpec((BN, nout), lambda i: (i, 0)),
        out_shape=jax.ShapeDtypeStruct((n, nout), jnp.float32),
    )(acc2, acc1, r2, wcat, bcat.reshape(1, nout))


def _make_edge_pass(width, nch, np_rows):
    rows_pt = np_rows // NS
    mesh = plsc.VectorSubcoreMesh(core_axis_name="c", subcore_axis_name="s")

    @functools.partial(
        pl.kernel, mesh=mesh,
        compiler_params=pltpu.CompilerParams(use_tc_tiling_on_sc=False),
        out_type=jax.ShapeDtypeStruct((NC, np_rows, width), jnp.float32),
        scratch_types=[
            pltpu.VMEM((nch, CHUNK), jnp.int32),
            pltpu.VMEM((nch, CHUNK), jnp.int32),
            pltpu.VMEM((CHUNK, width), jnp.float32),
            pltpu.VMEM((CHUNK, width), jnp.float32),
            pltpu.VMEM((16, width), jnp.float32),
            pltpu.VMEM_SHARED((np_rows, width), jnp.float32),
            pltpu.VMEM_SHARED((np_rows, width), jnp.float32),
            pltpu.SemaphoreType.DMA,
            pltpu.SemaphoreType.DMA,
        ],
    )
    def edge_pass(ytab, src_idx, dst_idx, out_acc,
                  src_v, dst_v, rows_a, rows_b, zbuf, acc_sh, tab_sh, sem_a, sem_b):
        cid = lax.axis_index("c")
        sid = lax.axis_index("s")
        wid = sid * NC + cid

        for co in (0, 16, width - 16):
            for r in range(16):
                zbuf[r, pl.ds(co, 16)] = jnp.zeros((16,), jnp.float32)
        row0 = sid * rows_pt

        def zloop(k, carry):
            pltpu.sync_copy(zbuf, acc_sh.at[pl.ds(row0 + k * 16, 16)])
            return carry
        lax.fori_loop(0, rows_pt // 16, zloop, 0)

        pltpu.sync_copy(src_idx.at[wid], src_v)
        pltpu.sync_copy(dst_idx.at[wid], dst_v)
        pltpu.sync_copy(ytab.at[pl.ds(row0, rows_pt)],
                        tab_sh.at[pl.ds(row0, rows_pt)])
        plsc.subcore_barrier()

        pltpu.async_copy(tab_sh.at[src_v.at[0]], rows_a, sem_a)

        def body(i, carry):
            j0 = 2 * i
            j1 = j0 + 1
            pltpu.async_copy(tab_sh.at[src_v.at[j1]], rows_b, sem_b)
            pltpu.make_async_copy(tab_sh.at[src_v.at[j0]], rows_a, sem_a).wait()
            pltpu.sync_copy(rows_a, acc_sh.at[dst_v.at[j0]], add=True)
            j2 = jnp.minimum(j0 + 2, nch - 1)
            pltpu.async_copy(tab_sh.at[src_v.at[j2]], rows_a, sem_a)
            pltpu.make_async_copy(ytab.at[src_v.at[j1]], rows_b, sem_b).wait()
            pltpu.sync_copy(rows_b, acc_sh.at[dst_v.at[j1]], add=True)
            return carry
        lax.fori_loop(0, nch // 2, body, 0)
        pltpu.make_async_copy(tab_sh.at[src_v.at[0]], rows_a, sem_a).wait()

        plsc.subcore_barrier()
        pltpu.sync_copy(acc_sh.at[pl.ds(row0, rows_pt)],
                        out_acc.at[cid, pl.ds(row0, rows_pt)])

    return edge_pass


def kernel(x, edge_index, Wl1, bl1, Wr1, Wl2, bl2, Wr2, Wh, bh, Wm, bm):
    n, d = x.shape
    e = edge_index.shape[1]
    nout = Wh.shape[0] + Wm.shape[0]

    epw = -(-e // (NW * 2 * CHUNK)) * 2 * CHUNK
    e_pad = NW * epw
    nch = epw // CHUNK
    np_rows = -(-(n + 1) // BN) * BN
    src = jnp.concatenate([edge_index[0], jnp.zeros((e_pad - e,), jnp.int32)])
    dst = jnp.concatenate([edge_index[1], jnp.full((e_pad - e,), n, jnp.int32)])
    src_r = src.reshape(NW, nch, CHUNK)
    dst_r = dst.reshape(NW, nch, CHUNK)
    xp = jnp.concatenate([x, jnp.zeros((np_rows - n, d), x.dtype)])

    w1 = jnp.concatenate([Wl1, Wr1], axis=0)
    w2 = jnp.concatenate([Wl2, Wr2], axis=0)
    wcat = jnp.concatenate([Wh, Wm], axis=0)
    bcat = jnp.concatenate([bh, bm])

    ytab, r1 = _tc_proj1(xp, w1, bl1, np_rows)
    acc1 = _make_edge_pass(W1COL, nch, np_rows)(ytab, src_r, dst_r)
    y2, r2 = _tc_mid(acc1, r1, w2, bl2, np_rows, np_rows)
    acc2 = _make_edge_pass(H, nch, np_rows)(y2, src_r, dst_r)
    out = _tc_head(acc2, acc1, r2, wcat, bcat, np_rows, nout)
    return out[:n, :Wh.shape[0]], out[:n, Wh.shape[0]:]


# speedup vs baseline: 1.7084x; 1.0141x over previous
import functools

import jax
import jax.numpy as jnp
from jax import lax
from jax.experimental import pallas as pl
from jax.experimental.pallas import tpu as pltpu
from jax.experimental.pallas import tpu_sc as plsc

H = 32
NC = 2
NS = 16
NW = NC * NS
CHUNK = 128
W1COL = 40
BN = 1280


def _tc_proj1(x, w1, bl1, n):
    def body(x_ref, w_ref, b_ref, yt_ref, r_ref):
        z = lax.dot_general(x_ref[...], w_ref[...], (((1,), (1,)), ((), ())),
                            preferred_element_type=jnp.float32)
        yt_ref[...] = jnp.concatenate(
            [z[:, :H], jnp.ones((BN, 1), jnp.float32),
             jnp.zeros((BN, W1COL - H - 1), jnp.float32)], axis=1)
        r_ref[...] = z[:, H:] + b_ref[...]

    d = x.shape[1]
    return pl.pallas_call(
        body,
        grid=(n // BN,),
        in_specs=[pl.BlockSpec((BN, d), lambda i: (i, 0)),
                  pl.BlockSpec((2 * H, d), lambda i: (0, 0)),
                  pl.BlockSpec((1, H), lambda i: (0, 0))],
        out_specs=[pl.BlockSpec((BN, W1COL), lambda i: (i, 0)),
                   pl.BlockSpec((BN, H), lambda i: (i, 0))],
        out_shape=[jax.ShapeDtypeStruct((n, W1COL), jnp.float32),
                   jax.ShapeDtypeStruct((n, H), jnp.float32)],
    )(x, w1, bl1.reshape(1, H))


def _tc_mid(acc1, r1, w2, bl2, n, np_rows):
    def body(a_ref, r1_ref, w_ref, b_ref, y2_ref, r2_ref):
        s = a_ref[0] + a_ref[1]
        c = jnp.maximum(s[:, H:H + 1], 1.0)
        h1 = jnp.maximum(s[:, :H] / c + r1_ref[...], 0.0)
        z = lax.dot_general(h1, w_ref[...], (((1,), (1,)), ((), ())),
                            preferred_element_type=jnp.float32)
        y2_ref[...] = z[:, :H]
        r2_ref[...] = z[:, H:] + b_ref[...]

    return pl.pallas_call(
        body,
        grid=(n // BN,),
        in_specs=[pl.BlockSpec((NC, BN, W1COL), lambda i: (0, i, 0)),
                  pl.BlockSpec((BN, H), lambda i: (i, 0)),
                  pl.BlockSpec((2 * H, H), lambda i: (0, 0)),
                  pl.BlockSpec((1, H), lambda i: (0, 0))],
        out_specs=[pl.BlockSpec((BN, H), lambda i: (i, 0)),
                   pl.BlockSpec((BN, H), lambda i: (i, 0))],
        out_shape=[jax.ShapeDtypeStruct((n, H), jnp.float32),
                   jax.ShapeDtypeStruct((n, H), jnp.float32)],
    )(acc1, r1, w2, bl2.reshape(1, H))


def _tc_head(acc2, acc1, r2, wcat, bcat, n, c_out, m_out):
    bn = 1000

    def body(a2_ref, a1_ref, r2_ref, w_ref, b_ref, oc_ref, om_ref):
        s2 = a2_ref[0] + a2_ref[1]
        cs = a1_ref[0] + a1_ref[1]
        c = jnp.maximum(cs[:, H:H + 1], 1.0)
        h2 = jnp.maximum(s2 / c + r2_ref[...], 0.0)
        z = lax.dot_general(h2, w_ref[...], (((1,), (1,)), ((), ())),
                            preferred_element_type=jnp.float32) + b_ref[...]
        oc_ref[...] = z[:, :c_out]
        om_ref[...] = z[:, c_out:]

    nout = c_out + m_out
    return pl.pallas_call(
        body,
        grid=(n // bn,),
        in_specs=[pl.BlockSpec((NC, bn, H), lambda i: (0, i, 0)),
                  pl.BlockSpec((NC, bn, W1COL), lambda i: (0, i, 0)),
                  pl.BlockSpec((bn, H), lambda i: (i, 0)),
                  pl.BlockSpec((nout, H), lambda i: (0, 0)),
                  pl.BlockSpec((1, nout), lambda i: (0, 0))],
        out_specs=[pl.BlockSpec((bn, c_out), lambda i: (i, 0)),
                   pl.BlockSpec((bn, m_out), lambda i: (i, 0))],
        out_shape=[jax.ShapeDtypeStruct((n, c_out), jnp.float32),
                   jax.ShapeDtypeStruct((n, m_out), jnp.float32)],
    )(acc2, acc1, r2, wcat, bcat.reshape(1, nout))


def _make_edge_pass(width, nch, np_rows):
    rows_pt = np_rows // NS
    mesh = plsc.VectorSubcoreMesh(core_axis_name="c", subcore_axis_name="s")

    @functools.partial(
        pl.kernel, mesh=mesh,
        compiler_params=pltpu.CompilerParams(use_tc_tiling_on_sc=False),
        out_type=jax.ShapeDtypeStruct((NC, np_rows, width), jnp.float32),
        scratch_types=[
            pltpu.VMEM((nch, CHUNK), jnp.int32),
            pltpu.VMEM((nch, CHUNK), jnp.int32),
            pltpu.VMEM((CHUNK, width), jnp.float32),
            pltpu.VMEM((CHUNK, width), jnp.float32),
            pltpu.VMEM((16, width), jnp.float32),
            pltpu.VMEM_SHARED((np_rows, width), jnp.float32),
            pltpu.VMEM_SHARED((np_rows, width), jnp.float32),
            pltpu.SemaphoreType.DMA,
            pltpu.SemaphoreType.DMA,
        ],
    )
    def edge_pass(ytab, src_idx, dst_idx, out_acc,
                  src_v, dst_v, rows_a, rows_b, zbuf, acc_sh, tab_sh, sem_a, sem_b):
        cid = lax.axis_index("c")
        sid = lax.axis_index("s")
        wid = sid * NC + cid

        for co in (0, 16, width - 16):
            for r in range(16):
                zbuf[r, pl.ds(co, 16)] = jnp.zeros((16,), jnp.float32)
        row0 = sid * rows_pt

        def zloop(k, carry):
            pltpu.sync_copy(zbuf, acc_sh.at[pl.ds(row0 + k * 16, 16)])
            return carry
        lax.fori_loop(0, rows_pt // 16, zloop, 0)

        pltpu.sync_copy(src_idx.at[wid], src_v)
        pltpu.sync_copy(dst_idx.at[wid], dst_v)
        pltpu.sync_copy(ytab.at[pl.ds(row0, rows_pt)],
                        tab_sh.at[pl.ds(row0, rows_pt)])
        plsc.subcore_barrier()

        pltpu.async_copy(tab_sh.at[src_v.at[0]], rows_a, sem_a)

        def body(i, carry):
            j0 = 2 * i
            j1 = j0 + 1
            pltpu.async_copy(tab_sh.at[src_v.at[j1]], rows_b, sem_b)
            pltpu.make_async_copy(tab_sh.at[src_v.at[j0]], rows_a, sem_a).wait()
            pltpu.sync_copy(rows_a, acc_sh.at[dst_v.at[j0]], add=True)
            j2 = jnp.minimum(j0 + 2, nch - 1)
            pltpu.async_copy(tab_sh.at[src_v.at[j2]], rows_a, sem_a)
            pltpu.make_async_copy(ytab.at[src_v.at[j1]], rows_b, sem_b).wait()
            pltpu.sync_copy(rows_b, acc_sh.at[dst_v.at[j1]], add=True)
            return carry
        lax.fori_loop(0, nch // 2, body, 0)
        pltpu.make_async_copy(tab_sh.at[src_v.at[0]], rows_a, sem_a).wait()

        plsc.subcore_barrier()
        pltpu.sync_copy(acc_sh.at[pl.ds(row0, rows_pt)],
                        out_acc.at[cid, pl.ds(row0, rows_pt)])

    return edge_pass


def kernel(x, edge_index, Wl1, bl1, Wr1, Wl2, bl2, Wr2, Wh, bh, Wm, bm):
    n, d = x.shape
    e = edge_index.shape[1]

    epw = -(-e // (NW * 2 * CHUNK)) * 2 * CHUNK
    e_pad = NW * epw
    nch = epw // CHUNK
    np_rows = -(-(n + 1) // BN) * BN
    src = jnp.concatenate([edge_index[0], jnp.zeros((e_pad - e,), jnp.int32)])
    dst = jnp.concatenate([edge_index[1], jnp.full((e_pad - e,), n, jnp.int32)])
    src_r = src.reshape(NW, nch, CHUNK)
    dst_r = dst.reshape(NW, nch, CHUNK)
    xp = jnp.concatenate([x, jnp.zeros((np_rows - n, d), x.dtype)])

    w1 = jnp.concatenate([Wl1, Wr1], axis=0)
    w2 = jnp.concatenate([Wl2, Wr2], axis=0)
    wcat = jnp.concatenate([Wh, Wm], axis=0)
    bcat = jnp.concatenate([bh, bm])

    ytab, r1 = _tc_proj1(xp, w1, bl1, np_rows)
    acc1 = _make_edge_pass(W1COL, nch, np_rows)(ytab, src_r, dst_r)
    y2, r2 = _tc_mid(acc1, r1, w2, bl2, np_rows, np_rows)
    acc2 = _make_edge_pass(H, nch, np_rows)(y2, src_r, dst_r)
    return _tc_head(acc2, acc1, r2, wcat, bcat, n, Wh.shape[0], Wm.shape[0])


# speedup vs baseline: 1.7869x; 1.0460x over previous
import functools

import jax
import jax.numpy as jnp
from jax import lax
from jax.experimental import pallas as pl
from jax.experimental.pallas import tpu as pltpu
from jax.experimental.pallas import tpu_sc as plsc

H = 32
NC = 2
NS = 16
NW = NC * NS
CHUNK = 128
W1COL = 40
BN = 1280


def _tc_proj1(x, w1, bl1, n):
    def body(x_ref, w_ref, b_ref, yt_ref, r_ref):
        z = lax.dot_general(x_ref[...], w_ref[...], (((1,), (1,)), ((), ())),
                            preferred_element_type=jnp.float32)
        yt_ref[...] = jnp.concatenate(
            [z[:, :H], jnp.ones((BN, 1), jnp.float32),
             jnp.zeros((BN, W1COL - H - 1), jnp.float32)], axis=1)
        r_ref[...] = z[:, H:] + b_ref[...]

    d = x.shape[1]
    return pl.pallas_call(
        body,
        grid=(n // BN,),
        in_specs=[pl.BlockSpec((BN, d), lambda i: (i, 0)),
                  pl.BlockSpec((2 * H, d), lambda i: (0, 0)),
                  pl.BlockSpec((1, H), lambda i: (0, 0))],
        out_specs=[pl.BlockSpec((BN, W1COL), lambda i: (i, 0)),
                   pl.BlockSpec((BN, H), lambda i: (i, 0))],
        out_shape=[jax.ShapeDtypeStruct((n, W1COL), jnp.float32),
                   jax.ShapeDtypeStruct((n, H), jnp.float32)],
    )(x, w1, bl1.reshape(1, H))


def _tc_mid(acc1, r1, w2, bl2, n, np_rows):
    def body(a_ref, r1_ref, w_ref, b_ref, y2_ref, r2_ref):
        s = a_ref[0] + a_ref[1]
        c = jnp.maximum(s[:, H:H + 1], 1.0)
        h1 = jnp.maximum(s[:, :H] / c + r1_ref[...], 0.0)
        z = lax.dot_general(h1, w_ref[...], (((1,), (1,)), ((), ())),
                            preferred_element_type=jnp.float32)
        y2_ref[...] = z[:, :H]
        r2_ref[...] = z[:, H:] + b_ref[...]

    return pl.pallas_call(
        body,
        grid=(n // BN,),
        in_specs=[pl.BlockSpec((NC, BN, W1COL), lambda i: (0, i, 0)),
                  pl.BlockSpec((BN, H), lambda i: (i, 0)),
                  pl.BlockSpec((2 * H, H), lambda i: (0, 0)),
                  pl.BlockSpec((1, H), lambda i: (0, 0))],
        out_specs=[pl.BlockSpec((BN, H), lambda i: (i, 0)),
                   pl.BlockSpec((BN, H), lambda i: (i, 0))],
        out_shape=[jax.ShapeDtypeStruct((n, H), jnp.float32),
                   jax.ShapeDtypeStruct((n, H), jnp.float32)],
    )(acc1, r1, w2, bl2.reshape(1, H))


def _tc_head(acc2, acc1, r2, wcat, bcat, n, c_out, m_out):
    bn = 1000

    def body(a2_ref, a1_ref, r2_ref, w_ref, b_ref, oc_ref, om_ref):
        s2 = a2_ref[0] + a2_ref[1]
        cs = a1_ref[0] + a1_ref[1]
        c = jnp.maximum(cs[:, H:H + 1], 1.0)
        h2 = jnp.maximum(s2 / c + r2_ref[...], 0.0)
        z = lax.dot_general(h2, w_ref[...], (((1,), (1,)), ((), ())),
                            preferred_element_type=jnp.float32) + b_ref[...]
        oc_ref[...] = z[:, :c_out]
        om_ref[...] = z[:, c_out:]

    nout = c_out + m_out
    return pl.pallas_call(
        body,
        grid=(n // bn,),
        in_specs=[pl.BlockSpec((NC, bn, H), lambda i: (0, i, 0)),
                  pl.BlockSpec((NC, bn, W1COL), lambda i: (0, i, 0)),
                  pl.BlockSpec((bn, H), lambda i: (i, 0)),
                  pl.BlockSpec((nout, H), lambda i: (0, 0)),
                  pl.BlockSpec((1, nout), lambda i: (0, 0))],
        out_specs=[pl.BlockSpec((bn, c_out), lambda i: (i, 0)),
                   pl.BlockSpec((bn, m_out), lambda i: (i, 0))],
        out_shape=[jax.ShapeDtypeStruct((n, c_out), jnp.float32),
                   jax.ShapeDtypeStruct((n, m_out), jnp.float32)],
    )(acc2, acc1, r2, wcat, bcat.reshape(1, nout))


def _make_edge_pass(width, nch, np_rows):
    rows_pt = np_rows // NS
    mesh = plsc.VectorSubcoreMesh(core_axis_name="c", subcore_axis_name="s")

    @functools.partial(
        pl.kernel, mesh=mesh,
        compiler_params=pltpu.CompilerParams(use_tc_tiling_on_sc=False),
        out_type=jax.ShapeDtypeStruct((NC, np_rows, width), jnp.float32),
        scratch_types=[
            pltpu.VMEM((nch, CHUNK), jnp.int32),
            pltpu.VMEM((nch, CHUNK), jnp.int32),
            [pltpu.VMEM((CHUNK, width), jnp.float32)] * 8,
            pltpu.VMEM((16, width), jnp.float32),
            pltpu.VMEM_SHARED((np_rows, width), jnp.float32),
            pltpu.VMEM_SHARED((np_rows, width), jnp.float32),
            [pltpu.SemaphoreType.DMA] * 8,
            [pltpu.SemaphoreType.DMA] * 8,
        ],
    )
    def edge_pass(ytab, src_idx, dst_idx, out_acc,
                  src_v, dst_v, rows, zbuf, acc_sh, tab_sh, sg, ss):
        cid = lax.axis_index("c")
        sid = lax.axis_index("s")
        wid = sid * NC + cid

        for co in (0, 16, width - 16):
            for r in range(16):
                zbuf[r, pl.ds(co, 16)] = jnp.zeros((16,), jnp.float32)
        row0 = sid * rows_pt

        def zloop(k, carry):
            pltpu.sync_copy(zbuf, acc_sh.at[pl.ds(row0 + k * 16, 16)])
            return carry
        lax.fori_loop(0, rows_pt // 16, zloop, 0)

        pltpu.sync_copy(src_idx.at[wid], src_v)
        pltpu.sync_copy(dst_idx.at[wid], dst_v)
        pltpu.sync_copy(ytab.at[pl.ds(row0, rows_pt)],
                        tab_sh.at[pl.ds(row0, rows_pt)])
        plsc.subcore_barrier()

        def g_issue(j, b):
            pltpu.async_copy(tab_sh.at[src_v.at[j]], rows[b], sg[b])

        def g_wait(b):
            pltpu.make_async_copy(tab_sh.at[src_v.at[0]], rows[b], sg[b]).wait()

        def s_issue(j, b):
            pltpu.async_copy(rows[b], acc_sh.at[dst_v.at[j]], ss[b], add=True)

        def s_wait(b):
            pltpu.make_async_copy(rows[b], acc_sh.at[dst_v.at[0]], ss[b]).wait()

        for b in range(4):
            g_issue(b, b)
        for b in range(8):
            if b >= 4:
                s_wait(b - 4)
            g_issue(b + 4, (b + 4) % 8)
            g_wait(b)
            s_issue(b, b)

        def body(g, carry):
            for b in range(8):
                j = 8 * g + b
                s_wait((b + 4) % 8)
                g_issue(jnp.minimum(j + 4, nch - 1), (b + 4) % 8)
                g_wait(b)
                s_issue(j, b)
            return carry
        lax.fori_loop(1, nch // 8, body, 0)

        for b in range(4):
            g_wait(b)
        for b in range(4, 8):
            s_wait(b)

        plsc.subcore_barrier()
        pltpu.sync_copy(acc_sh.at[pl.ds(row0, rows_pt)],
                        out_acc.at[cid, pl.ds(row0, rows_pt)])

    return edge_pass


def kernel(x, edge_index, Wl1, bl1, Wr1, Wl2, bl2, Wr2, Wh, bh, Wm, bm):
    n, d = x.shape
    e = edge_index.shape[1]

    epw = -(-e // (NW * 8 * CHUNK)) * 8 * CHUNK
    e_pad = NW * epw
    nch = epw // CHUNK
    np_rows = -(-(n + 1) // BN) * BN
    src = jnp.concatenate([edge_index[0], jnp.zeros((e_pad - e,), jnp.int32)])
    dst = jnp.concatenate([edge_index[1], jnp.full((e_pad - e,), n, jnp.int32)])
    src_r = src.reshape(NW, nch, CHUNK)
    dst_r = dst.reshape(NW, nch, CHUNK)
    xp = jnp.concatenate([x, jnp.zeros((np_rows - n, d), x.dtype)])

    w1 = jnp.concatenate([Wl1, Wr1], axis=0)
    w2 = jnp.concatenate([Wl2, Wr2], axis=0)
    wcat = jnp.concatenate([Wh, Wm], axis=0)
    bcat = jnp.concatenate([bh, bm])

    ytab, r1 = _tc_proj1(xp, w1, bl1, np_rows)
    acc1 = _make_edge_pass(W1COL, nch, np_rows)(ytab, src_r, dst_r)
    y2, r2 = _tc_mid(acc1, r1, w2, bl2, np_rows, np_rows)
    acc2 = _make_edge_pass(H, nch, np_rows)(y2, src_r, dst_r)
    return _tc_head(acc2, acc1, r2, wcat, bcat, n, Wh.shape[0], Wm.shape[0])


# speedup vs baseline: 1.7871x; 1.0001x over previous
import functools

import jax
import jax.numpy as jnp
from jax import lax
from jax.experimental import pallas as pl
from jax.experimental.pallas import tpu as pltpu
from jax.experimental.pallas import tpu_sc as plsc

H = 32
NC = 2
NS = 16
NW = NC * NS
CHUNK = 128
W1COL = 40
BN = 1280


def _tc_proj1(x, w1, bl1, n):
    def body(x_ref, w_ref, b_ref, yt_ref, r_ref):
        z = lax.dot_general(x_ref[...], w_ref[...], (((1,), (1,)), ((), ())),
                            preferred_element_type=jnp.float32)
        yt_ref[...] = jnp.concatenate(
            [z[:, :H], jnp.ones((BN, 1), jnp.float32),
             jnp.zeros((BN, W1COL - H - 1), jnp.float32)], axis=1)
        r_ref[...] = z[:, H:] + b_ref[...]

    d = x.shape[1]
    return pl.pallas_call(
        body,
        grid=(n // BN,),
        in_specs=[pl.BlockSpec((BN, d), lambda i: (i, 0)),
                  pl.BlockSpec((2 * H, d), lambda i: (0, 0)),
                  pl.BlockSpec((1, H), lambda i: (0, 0))],
        out_specs=[pl.BlockSpec((BN, W1COL), lambda i: (i, 0)),
                   pl.BlockSpec((BN, H), lambda i: (i, 0))],
        out_shape=[jax.ShapeDtypeStruct((n, W1COL), jnp.float32),
                   jax.ShapeDtypeStruct((n, H), jnp.float32)],
    )(x, w1, bl1.reshape(1, H))


def _tc_mid(acc1, r1, w2, bl2, n):
    def body(a_ref, r1_ref, w_ref, b_ref, y2_ref, r2_ref):
        s = a_ref[0] + a_ref[1]
        c = jnp.maximum(s[:, H:H + 1], 1.0)
        h1 = jnp.maximum(s[:, :H] / c + r1_ref[...], 0.0)
        z = lax.dot_general(h1, w_ref[...], (((1,), (1,)), ((), ())),
                            preferred_element_type=jnp.float32)
        y2_ref[...] = z[:, :H]
        r2_ref[...] = z[:, H:] + b_ref[...]

    return pl.pallas_call(
        body,
        grid=(n // BN,),
        in_specs=[pl.BlockSpec((NC, BN, W1COL), lambda i: (0, i, 0)),
                  pl.BlockSpec((BN, H), lambda i: (i, 0)),
                  pl.BlockSpec((2 * H, H), lambda i: (0, 0)),
                  pl.BlockSpec((1, H), lambda i: (0, 0))],
        out_specs=[pl.BlockSpec((BN, H), lambda i: (i, 0)),
                   pl.BlockSpec((BN, H), lambda i: (i, 0))],
        out_shape=[jax.ShapeDtypeStruct((n, H), jnp.float32),
                   jax.ShapeDtypeStruct((n, H), jnp.float32)],
    )(acc1, r1, w2, bl2.reshape(1, H))


def _tc_head(acc2, acc1, r2, wcat, bcat, n, c_out, m_out):
    bn = 1000

    def body(a2_ref, a1_ref, r2_ref, w_ref, b_ref, oc_ref, om_ref):
        s2 = a2_ref[0] + a2_ref[1]
        cs = a1_ref[0] + a1_ref[1]
        c = jnp.maximum(cs[:, H:H + 1], 1.0)
        h2 = jnp.maximum(s2 / c + r2_ref[...], 0.0)
        z = lax.dot_general(h2, w_ref[...], (((1,), (1,)), ((), ())),
                            preferred_element_type=jnp.float32) + b_ref[...]
        oc_ref[...] = z[:, :c_out]
        om_ref[...] = z[:, c_out:]

    nout = c_out + m_out
    return pl.pallas_call(
        body,
        grid=(n // bn,),
        in_specs=[pl.BlockSpec((NC, bn, H), lambda i: (0, i, 0)),
                  pl.BlockSpec((NC, bn, W1COL), lambda i: (0, i, 0)),
                  pl.BlockSpec((bn, H), lambda i: (i, 0)),
                  pl.BlockSpec((nout, H), lambda i: (0, 0)),
                  pl.BlockSpec((1, nout), lambda i: (0, 0))],
        out_specs=[pl.BlockSpec((bn, c_out), lambda i: (i, 0)),
                   pl.BlockSpec((bn, m_out), lambda i: (i, 0))],
        out_shape=[jax.ShapeDtypeStruct((n, c_out), jnp.float32),
                   jax.ShapeDtypeStruct((n, m_out), jnp.float32)],
    )(acc2, acc1, r2, wcat, bcat.reshape(1, nout))


def _make_edge_pass(width, nch, np_rows):
    rows_pt = np_rows // NS
    mesh = plsc.VectorSubcoreMesh(core_axis_name="c", subcore_axis_name="s")

    @functools.partial(
        pl.kernel, mesh=mesh,
        compiler_params=pltpu.CompilerParams(use_tc_tiling_on_sc=False),
        out_type=jax.ShapeDtypeStruct((NC, np_rows, width), jnp.float32),
        scratch_types=[
            pltpu.VMEM((nch, CHUNK), jnp.int32),
            pltpu.VMEM((nch, CHUNK), jnp.int32),
            [pltpu.VMEM((CHUNK, width), jnp.float32)] * 8,
            pltpu.VMEM((16, width), jnp.float32),
            pltpu.VMEM_SHARED((np_rows, width), jnp.float32),
            pltpu.VMEM_SHARED((np_rows, width), jnp.float32),
            [pltpu.SemaphoreType.DMA] * 8,
            [pltpu.SemaphoreType.DMA] * 8,
        ],
    )
    def edge_pass(ytab, src_idx, dst_idx, out_acc,
                  src_v, dst_v, rows, zbuf, acc_sh, tab_sh, sg, ss):
        cid = lax.axis_index("c")
        sid = lax.axis_index("s")
        wid = sid * NC + cid

        for co in (0, 16, width - 16):
            for r in range(16):
                zbuf[r, pl.ds(co, 16)] = jnp.zeros((16,), jnp.float32)
        row0 = sid * rows_pt

        def zloop(k, carry):
            pltpu.sync_copy(zbuf, acc_sh.at[pl.ds(row0 + k * 16, 16)])
            return carry
        lax.fori_loop(0, rows_pt // 16, zloop, 0)

        pltpu.sync_copy(src_idx.at[wid], src_v)
        pltpu.sync_copy(dst_idx.at[wid], dst_v)
        pltpu.sync_copy(ytab.at[pl.ds(row0, rows_pt)],
                        tab_sh.at[pl.ds(row0, rows_pt)])
        plsc.subcore_barrier()

        def g_issue(j, b):
            pltpu.async_copy(tab_sh.at[src_v.at[j]], rows[b], sg[b])

        def g_wait(b):
            pltpu.make_async_copy(tab_sh.at[src_v.at[0]], rows[b], sg[b]).wait()

        def s_issue(j, b):
            pltpu.async_copy(rows[b], acc_sh.at[dst_v.at[j]], ss[b], add=True)

        def s_wait(b):
            pltpu.make_async_copy(rows[b], acc_sh.at[dst_v.at[0]], ss[b]).wait()

        for b in range(4):
            g_issue(b, b)
        for b in range(8):
            if b >= 4:
                s_wait(b - 4)
            g_issue(b + 4, (b + 4) % 8)
            g_wait(b)
            s_issue(b, b)

        def body(g, carry):
            for b in range(8):
                j = 8 * g + b
                s_wait((b + 4) % 8)
                g_issue(jnp.minimum(j + 4, nch - 1), (b + 4) % 8)
                g_wait(b)
                s_issue(j, b)
            return carry
        lax.fori_loop(1, nch // 8, body, 0)

        for b in range(4):
            g_wait(b)
        for b in range(4, 8):
            s_wait(b)

        plsc.subcore_barrier()
        pltpu.sync_copy(acc_sh.at[pl.ds(row0, rows_pt)],
                        out_acc.at[cid, pl.ds(row0, rows_pt)])

    return edge_pass


def kernel(x, edge_index, Wl1, bl1, Wr1, Wl2, bl2, Wr2, Wh, bh, Wm, bm):
    n, d = x.shape
    e = edge_index.shape[1]

    epw = -(-e // (NW * 8 * CHUNK)) * 8 * CHUNK
    e_pad = NW * epw
    nch = epw // CHUNK
    np_rows = -(-(n + 1) // BN) * BN
    src = jnp.concatenate([edge_index[0], jnp.zeros((e_pad - e,), jnp.int32)])
    dst = jnp.concatenate([edge_index[1], jnp.full((e_pad - e,), n, jnp.int32)])
    src_r = src.reshape(NW, nch, CHUNK)
    dst_r = dst.reshape(NW, nch, CHUNK)
    xp = jnp.concatenate([x, jnp.zeros((np_rows - n, d), x.dtype)])

    w1 = jnp.concatenate([Wl1, Wr1], axis=0)
    w2 = jnp.concatenate([Wl2, Wr2], axis=0)
    wcat = jnp.concatenate([Wh, Wm], axis=0)
    bcat = jnp.concatenate([bh, bm])

    ytab, r1 = _tc_proj1(xp, w1, bl1, np_rows)
    acc1 = _make_edge_pass(W1COL, nch, np_rows)(ytab, src_r, dst_r)
    y2, r2 = _tc_mid(acc1, r1, w2, bl2, np_rows)
    acc2 = _make_edge_pass(H, nch, np_rows)(y2, src_r, dst_r)
    return _tc_head(acc2, acc1, r2, wcat, bcat, n, Wh.shape[0], Wm.shape[0])


# speedup vs baseline: 1.8557x; 1.0384x over previous
import functools

import jax
import jax.numpy as jnp
from jax import lax
from jax.experimental import pallas as pl
from jax.experimental.pallas import tpu as pltpu
from jax.experimental.pallas import tpu_sc as plsc

H = 32
NC = 2
NS = 16
NW = NC * NS
CHUNK = 128
W1COL = 40
BN = 2560


def _tc_proj1(x, w1, bl1, n):
    def body(x_ref, w_ref, b_ref, yt_ref, r_ref):
        z = lax.dot_general(x_ref[...], w_ref[...], (((1,), (1,)), ((), ())),
                            preferred_element_type=jnp.float32)
        yt_ref[...] = jnp.concatenate(
            [z[:, :H], jnp.ones((BN, 1), jnp.float32),
             jnp.zeros((BN, W1COL - H - 1), jnp.float32)], axis=1)
        r_ref[...] = z[:, H:] + b_ref[...]

    d = x.shape[1]
    return pl.pallas_call(
        body,
        grid=(n // BN,),
        in_specs=[pl.BlockSpec((BN, d), lambda i: (i, 0)),
                  pl.BlockSpec((2 * H, d), lambda i: (0, 0)),
                  pl.BlockSpec((1, H), lambda i: (0, 0))],
        out_specs=[pl.BlockSpec((BN, W1COL), lambda i: (i, 0)),
                   pl.BlockSpec((BN, H), lambda i: (i, 0))],
        out_shape=[jax.ShapeDtypeStruct((n, W1COL), jnp.float32),
                   jax.ShapeDtypeStruct((n, H), jnp.float32)],
    )(x, w1, bl1.reshape(1, H))


def _tc_mid(acc1, r1, w2, bl2, n):
    def body(a_ref, r1_ref, w_ref, b_ref, y2_ref, r2_ref):
        s = a_ref[0] + a_ref[1]
        c = jnp.maximum(s[:, H:H + 1], 1.0)
        h1 = jnp.maximum(s[:, :H] / c + r1_ref[...], 0.0)
        z = lax.dot_general(h1, w_ref[...], (((1,), (1,)), ((), ())),
                            preferred_element_type=jnp.float32)
        y2_ref[...] = z[:, :H]
        r2_ref[...] = z[:, H:] + b_ref[...]

    return pl.pallas_call(
        body,
        grid=(n // BN,),
        in_specs=[pl.BlockSpec((NC, BN, W1COL), lambda i: (0, i, 0)),
                  pl.BlockSpec((BN, H), lambda i: (i, 0)),
                  pl.BlockSpec((2 * H, H), lambda i: (0, 0)),
                  pl.BlockSpec((1, H), lambda i: (0, 0))],
        out_specs=[pl.BlockSpec((BN, H), lambda i: (i, 0)),
                   pl.BlockSpec((BN, H), lambda i: (i, 0))],
        out_shape=[jax.ShapeDtypeStruct((n, H), jnp.float32),
                   jax.ShapeDtypeStruct((n, H), jnp.float32)],
    )(acc1, r1, w2, bl2.reshape(1, H))


def _tc_head(acc2, acc1, r2, wcat, bcat, n, c_out, m_out):
    bn = 2000

    def body(a2_ref, a1_ref, r2_ref, w_ref, b_ref, oc_ref, om_ref):
        s2 = a2_ref[0] + a2_ref[1]
        cs = a1_ref[0] + a1_ref[1]
        c = jnp.maximum(cs[:, H:H + 1], 1.0)
        h2 = jnp.maximum(s2 / c + r2_ref[...], 0.0)
        z = lax.dot_general(h2, w_ref[...], (((1,), (1,)), ((), ())),
                            preferred_element_type=jnp.float32) + b_ref[...]
        oc_ref[...] = z[:, :c_out]
        om_ref[...] = z[:, c_out:]

    nout = c_out + m_out
    return pl.pallas_call(
        body,
        grid=(n // bn,),
        in_specs=[pl.BlockSpec((NC, bn, H), lambda i: (0, i, 0)),
                  pl.BlockSpec((NC, bn, W1COL), lambda i: (0, i, 0)),
                  pl.BlockSpec((bn, H), lambda i: (i, 0)),
                  pl.BlockSpec((nout, H), lambda i: (0, 0)),
                  pl.BlockSpec((1, nout), lambda i: (0, 0))],
        out_specs=[pl.BlockSpec((bn, c_out), lambda i: (i, 0)),
                   pl.BlockSpec((bn, m_out), lambda i: (i, 0))],
        out_shape=[jax.ShapeDtypeStruct((n, c_out), jnp.float32),
                   jax.ShapeDtypeStruct((n, m_out), jnp.float32)],
    )(acc2, acc1, r2, wcat, bcat.reshape(1, nout))


def _make_edge_pass(width, nch, np_rows):
    rows_pt = np_rows // NS
    mesh = plsc.VectorSubcoreMesh(core_axis_name="c", subcore_axis_name="s")

    @functools.partial(
        pl.kernel, mesh=mesh,
        compiler_params=pltpu.CompilerParams(use_tc_tiling_on_sc=False),
        out_type=jax.ShapeDtypeStruct((NC, np_rows, width), jnp.float32),
        scratch_types=[
            pltpu.VMEM((nch, CHUNK), jnp.int32),
            pltpu.VMEM((nch, CHUNK), jnp.int32),
            [pltpu.VMEM((CHUNK, width), jnp.float32)] * 8,
            pltpu.VMEM((16, width), jnp.float32),
            pltpu.VMEM_SHARED((np_rows, width), jnp.float32),
            pltpu.VMEM_SHARED((np_rows, width), jnp.float32),
            [pltpu.SemaphoreType.DMA] * 8,
            [pltpu.SemaphoreType.DMA] * 8,
        ],
    )
    def edge_pass(ytab, src_idx, dst_idx, out_acc,
                  src_v, dst_v, rows, zbuf, acc_sh, tab_sh, sg, ss):
        cid = lax.axis_index("c")
        sid = lax.axis_index("s")
        wid = sid * NC + cid

        for co in (0, 16, width - 16):
            for r in range(16):
                zbuf[r, pl.ds(co, 16)] = jnp.zeros((16,), jnp.float32)
        row0 = sid * rows_pt

        def zloop(k, carry):
            pltpu.sync_copy(zbuf, acc_sh.at[pl.ds(row0 + k * 16, 16)])
            return carry
        lax.fori_loop(0, rows_pt // 16, zloop, 0)

        pltpu.sync_copy(src_idx.at[wid], src_v)
        pltpu.sync_copy(dst_idx.at[wid], dst_v)
        pltpu.sync_copy(ytab.at[pl.ds(row0, rows_pt)],
                        tab_sh.at[pl.ds(row0, rows_pt)])
        plsc.subcore_barrier()

        def g_issue(j, b):
            pltpu.async_copy(tab_sh.at[src_v.at[j]], rows[b], sg[b])

        def g_wait(b):
            pltpu.make_async_copy(tab_sh.at[src_v.at[0]], rows[b], sg[b]).wait()

        def s_issue(j, b):
            pltpu.async_copy(rows[b], acc_sh.at[dst_v.at[j]], ss[b], add=True)

        def s_wait(b):
            pltpu.make_async_copy(rows[b], acc_sh.at[dst_v.at[0]], ss[b]).wait()

        for b in range(4):
            g_issue(b, b)
        for b in range(8):
            if b >= 4:
                s_wait(b - 4)
            g_issue(b + 4, (b + 4) % 8)
            g_wait(b)
            s_issue(b, b)

        def body(g, carry):
            for b in range(8):
                j = 8 * g + b
                s_wait((b + 4) % 8)
                g_issue(jnp.minimum(j + 4, nch - 1), (b + 4) % 8)
                g_wait(b)
                s_issue(j, b)
            return carry
        lax.fori_loop(1, nch // 8, body, 0)

        for b in range(4):
            g_wait(b)
        for b in range(4, 8):
            s_wait(b)

        plsc.subcore_barrier()
        pltpu.sync_copy(acc_sh.at[pl.ds(row0, rows_pt)],
                        out_acc.at[cid, pl.ds(row0, rows_pt)])

    return edge_pass


def kernel(x, edge_index, Wl1, bl1, Wr1, Wl2, bl2, Wr2, Wh, bh, Wm, bm):
    n, d = x.shape
    e = edge_index.shape[1]

    epw = -(-e // (NW * 8 * CHUNK)) * 8 * CHUNK
    e_pad = NW * epw
    nch = epw // CHUNK
    np_rows = -(-(n + 1) // BN) * BN
    src = jnp.concatenate([edge_index[0], jnp.zeros((e_pad - e,), jnp.int32)])
    dst = jnp.concatenate([edge_index[1], jnp.full((e_pad - e,), n, jnp.int32)])
    src_r = src.reshape(NW, nch, CHUNK)
    dst_r = dst.reshape(NW, nch, CHUNK)
    xp = jnp.concatenate([x, jnp.zeros((np_rows - n, d), x.dtype)])

    w1 = jnp.concatenate([Wl1, Wr1], axis=0)
    w2 = jnp.concatenate([Wl2, Wr2], axis=0)
    wcat = jnp.concatenate([Wh, Wm], axis=0)
    bcat = jnp.concatenate([bh, bm])

    ytab, r1 = _tc_proj1(xp, w1, bl1, np_rows)
    acc1 = _make_edge_pass(W1COL, nch, np_rows)(ytab, src_r, dst_r)
    y2, r2 = _tc_mid(acc1, r1, w2, bl2, np_rows)
    acc2 = _make_edge_pass(H, nch, np_rows)(y2, src_r, dst_r)
    return _tc_head(acc2, acc1, r2, wcat, bcat, n, Wh.shape[0], Wm.shape[0])


# speedup vs baseline: 1.8698x; 1.0076x over previous
import functools

import jax
import jax.numpy as jnp
from jax import lax
from jax.experimental import pallas as pl
from jax.experimental.pallas import tpu as pltpu
from jax.experimental.pallas import tpu_sc as plsc

H = 32
NC = 2
NS = 16
NW = NC * NS
CHUNK = 128
W1COL = 40
BN = 2560


def _tc_proj1(x, w1, bl1, n):
    def body(x_ref, w_ref, b_ref, yt_ref, r_ref):
        z = lax.dot_general(x_ref[...], w_ref[...], (((1,), (1,)), ((), ())),
                            preferred_element_type=jnp.float32)
        yt_ref[...] = jnp.concatenate(
            [z[:, :H], jnp.ones((BN, 1), jnp.float32),
             jnp.zeros((BN, W1COL - H - 1), jnp.float32)], axis=1)
        r_ref[...] = z[:, H:] + b_ref[...]

    d = x.shape[1]
    return pl.pallas_call(
        body,
        grid=(n // BN,),
        in_specs=[pl.BlockSpec((BN, d), lambda i: (i, 0)),
                  pl.BlockSpec((2 * H, d), lambda i: (0, 0)),
                  pl.BlockSpec((1, H), lambda i: (0, 0))],
        out_specs=[pl.BlockSpec((BN, W1COL), lambda i: (i, 0)),
                   pl.BlockSpec((BN, H), lambda i: (i, 0))],
        out_shape=[jax.ShapeDtypeStruct((n, W1COL), jnp.float32),
                   jax.ShapeDtypeStruct((n, H), jnp.float32)],
    )(x, w1, bl1.reshape(1, H))


def _tc_mid(acc1, r1, w2, bl2, n):
    def body(a_ref, r1_ref, w_ref, b_ref, y2_ref, r2_ref):
        s = a_ref[0] + a_ref[1]
        c = jnp.maximum(s[:, H:H + 1], 1.0)
        h1 = jnp.maximum(s[:, :H] / c + r1_ref[...], 0.0)
        z = lax.dot_general(h1, w_ref[...], (((1,), (1,)), ((), ())),
                            preferred_element_type=jnp.float32)
        y2_ref[...] = z[:, :H]
        r2_ref[...] = z[:, H:] + b_ref[...]

    return pl.pallas_call(
        body,
        grid=(n // BN,),
        in_specs=[pl.BlockSpec((NC, BN, W1COL), lambda i: (0, i, 0)),
                  pl.BlockSpec((BN, H), lambda i: (i, 0)),
                  pl.BlockSpec((2 * H, H), lambda i: (0, 0)),
                  pl.BlockSpec((1, H), lambda i: (0, 0))],
        out_specs=[pl.BlockSpec((BN, H), lambda i: (i, 0)),
                   pl.BlockSpec((BN, H), lambda i: (i, 0))],
        out_shape=[jax.ShapeDtypeStruct((n, H), jnp.float32),
                   jax.ShapeDtypeStruct((n, H), jnp.float32)],
    )(acc1, r1, w2, bl2.reshape(1, H))


def _tc_head(acc2, acc1, r2, wcat, bcat, n, c_out, m_out):
    bn = 2000

    def body(a2_ref, a1_ref, r2_ref, w_ref, b_ref, oc_ref, om_ref):
        s2 = a2_ref[0] + a2_ref[1]
        cs = a1_ref[0] + a1_ref[1]
        c = jnp.maximum(cs[:, H:H + 1], 1.0)
        h2 = jnp.maximum(s2 / c + r2_ref[...], 0.0)
        z = lax.dot_general(h2, w_ref[...], (((1,), (1,)), ((), ())),
                            preferred_element_type=jnp.float32) + b_ref[...]
        oc_ref[...] = z[:, :c_out]
        om_ref[...] = z[:, c_out:]

    nout = c_out + m_out
    return pl.pallas_call(
        body,
        grid=(n // bn,),
        in_specs=[pl.BlockSpec((NC, bn, H), lambda i: (0, i, 0)),
                  pl.BlockSpec((NC, bn, W1COL), lambda i: (0, i, 0)),
                  pl.BlockSpec((bn, H), lambda i: (i, 0)),
                  pl.BlockSpec((nout, H), lambda i: (0, 0)),
                  pl.BlockSpec((1, nout), lambda i: (0, 0))],
        out_specs=[pl.BlockSpec((bn, c_out), lambda i: (i, 0)),
                   pl.BlockSpec((bn, m_out), lambda i: (i, 0))],
        out_shape=[jax.ShapeDtypeStruct((n, c_out), jnp.float32),
                   jax.ShapeDtypeStruct((n, m_out), jnp.float32)],
    )(acc2, acc1, r2, wcat, bcat.reshape(1, nout))


def _make_edge_pass(width, nch, np_rows):
    rows_pt = np_rows // NS
    mesh = plsc.VectorSubcoreMesh(core_axis_name="c", subcore_axis_name="s")

    @functools.partial(
        pl.kernel, mesh=mesh,
        compiler_params=pltpu.CompilerParams(use_tc_tiling_on_sc=False),
        out_type=jax.ShapeDtypeStruct((NC, np_rows, width), jnp.float32),
        scratch_types=[
            pltpu.VMEM((nch, CHUNK), jnp.int32),
            pltpu.VMEM((nch, CHUNK), jnp.int32),
            [pltpu.VMEM((CHUNK, width), jnp.float32)] * 10,
            pltpu.VMEM((16, width), jnp.float32),
            pltpu.VMEM_SHARED((np_rows, width), jnp.float32),
            pltpu.VMEM_SHARED((np_rows, width), jnp.float32),
            [pltpu.SemaphoreType.DMA] * 10,
            [pltpu.SemaphoreType.DMA] * 10,
        ],
    )
    def edge_pass(ytab, src_idx, dst_idx, out_acc,
                  src_v, dst_v, rows, zbuf, acc_sh, tab_sh, sg, ss):
        cid = lax.axis_index("c")
        sid = lax.axis_index("s")
        wid = sid * NC + cid

        for co in (0, 16, width - 16):
            for r in range(16):
                zbuf[r, pl.ds(co, 16)] = jnp.zeros((16,), jnp.float32)
        row0 = sid * rows_pt

        def zloop(k, carry):
            pltpu.sync_copy(zbuf, acc_sh.at[pl.ds(row0 + k * 16, 16)])
            return carry
        lax.fori_loop(0, rows_pt // 16, zloop, 0)

        pltpu.sync_copy(src_idx.at[wid], src_v)
        pltpu.sync_copy(dst_idx.at[wid], dst_v)
        pltpu.sync_copy(ytab.at[pl.ds(row0, rows_pt)],
                        tab_sh.at[pl.ds(row0, rows_pt)])
        plsc.subcore_barrier()

        def g_issue(j, b):
            pltpu.async_copy(tab_sh.at[src_v.at[j]], rows[b], sg[b])

        def g_wait(b):
            pltpu.make_async_copy(tab_sh.at[src_v.at[0]], rows[b], sg[b]).wait()

        def s_issue(j, b):
            pltpu.async_copy(rows[b], acc_sh.at[dst_v.at[j]], ss[b], add=True)

        def s_wait(b):
            pltpu.make_async_copy(rows[b], acc_sh.at[dst_v.at[0]], ss[b]).wait()

        nb, look = 10, 5
        for b in range(look):
            g_issue(b, b)
        for b in range(nb):
            if b >= look:
                s_wait(b - look)
            g_issue(b + look, (b + look) % nb)
            g_wait(b)
            s_issue(b, b)

        def body(g, carry):
            for b in range(nb):
                j = nb * g + b
                s_wait((b + look) % nb)
                g_issue(jnp.minimum(j + look, nch - 1), (b + look) % nb)
                g_wait(b)
                s_issue(j, b)
            return carry
        lax.fori_loop(1, nch // nb, body, 0)

        for b in range(look):
            g_wait(b)
        for b in range(nb - look, nb):
            s_wait(b)

        plsc.subcore_barrier()
        pltpu.sync_copy(acc_sh.at[pl.ds(row0, rows_pt)],
                        out_acc.at[cid, pl.ds(row0, rows_pt)])

    return edge_pass


def kernel(x, edge_index, Wl1, bl1, Wr1, Wl2, bl2, Wr2, Wh, bh, Wm, bm):
    n, d = x.shape
    e = edge_index.shape[1]

    epw = -(-e // (NW * 10 * CHUNK)) * 10 * CHUNK
    e_pad = NW * epw
    nch = epw // CHUNK
    np_rows = -(-(n + 1) // BN) * BN
    src = jnp.concatenate([edge_index[0], jnp.zeros((e_pad - e,), jnp.int32)])
    dst = jnp.concatenate([edge_index[1], jnp.full((e_pad - e,), n, jnp.int32)])
    src_r = src.reshape(NW, nch, CHUNK)
    dst_r = dst.reshape(NW, nch, CHUNK)
    xp = jnp.concatenate([x, jnp.zeros((np_rows - n, d), x.dtype)])

    w1 = jnp.concatenate([Wl1, Wr1], axis=0)
    w2 = jnp.concatenate([Wl2, Wr2], axis=0)
    wcat = jnp.concatenate([Wh, Wm], axis=0)
    bcat = jnp.concatenate([bh, bm])

    ytab, r1 = _tc_proj1(xp, w1, bl1, np_rows)
    acc1 = _make_edge_pass(W1COL, nch, np_rows)(ytab, src_r, dst_r)
    y2, r2 = _tc_mid(acc1, r1, w2, bl2, np_rows)
    acc2 = _make_edge_pass(H, nch, np_rows)(y2, src_r, dst_r)
    return _tc_head(acc2, acc1, r2, wcat, bcat, n, Wh.shape[0], Wm.shape[0])


# speedup vs baseline: 1.8844x; 1.0078x over previous
import functools

import jax
import jax.numpy as jnp
from jax import lax
from jax.experimental import pallas as pl
from jax.experimental.pallas import tpu as pltpu
from jax.experimental.pallas import tpu_sc as plsc

H = 32
NC = 2
NS = 16
NW = NC * NS
CHUNK = 128
W1COL = 40
BN = 2560


def _tc_proj1(x, w1, bl1, n, np_rows):
    BN1 = 2000

    def body(x_ref, w_ref, b_ref, yt_ref, r_ref):
        z = lax.dot_general(x_ref[...], w_ref[...], (((1,), (1,)), ((), ())),
                            preferred_element_type=jnp.float32)
        yt_ref[...] = jnp.concatenate(
            [z[:, :H], jnp.ones((BN1, 1), jnp.float32),
             jnp.zeros((BN1, W1COL - H - 1), jnp.float32)], axis=1)
        r_ref[...] = z[:, H:] + b_ref[...]

    d = x.shape[1]
    return pl.pallas_call(
        body,
        grid=(n // BN1,),
        in_specs=[pl.BlockSpec((BN1, d), lambda i: (i, 0)),
                  pl.BlockSpec((2 * H, d), lambda i: (0, 0)),
                  pl.BlockSpec((1, H), lambda i: (0, 0))],
        out_specs=[pl.BlockSpec((BN1, W1COL), lambda i: (i, 0)),
                   pl.BlockSpec((BN1, H), lambda i: (i, 0))],
        out_shape=[jax.ShapeDtypeStruct((np_rows, W1COL), jnp.float32),
                   jax.ShapeDtypeStruct((np_rows, H), jnp.float32)],
    )(x, w1, bl1.reshape(1, H))


def _tc_mid(acc1, r1, w2, bl2, n):
    def body(a_ref, r1_ref, w_ref, b_ref, y2_ref, r2_ref):
        s = a_ref[0] + a_ref[1]
        c = jnp.maximum(s[:, H:H + 1], 1.0)
        h1 = jnp.maximum(s[:, :H] / c + r1_ref[...], 0.0)
        z = lax.dot_general(h1, w_ref[...], (((1,), (1,)), ((), ())),
                            preferred_element_type=jnp.float32)
        y2_ref[...] = z[:, :H]
        r2_ref[...] = z[:, H:] + b_ref[...]

    return pl.pallas_call(
        body,
        grid=(n // BN,),
        in_specs=[pl.BlockSpec((NC, BN, W1COL), lambda i: (0, i, 0)),
                  pl.BlockSpec((BN, H), lambda i: (i, 0)),
                  pl.BlockSpec((2 * H, H), lambda i: (0, 0)),
                  pl.BlockSpec((1, H), lambda i: (0, 0))],
        out_specs=[pl.BlockSpec((BN, H), lambda i: (i, 0)),
                   pl.BlockSpec((BN, H), lambda i: (i, 0))],
        out_shape=[jax.ShapeDtypeStruct((n, H), jnp.float32),
                   jax.ShapeDtypeStruct((n, H), jnp.float32)],
    )(acc1, r1, w2, bl2.reshape(1, H))


def _tc_head(acc2, acc1, r2, wcat, bcat, n, c_out, m_out):
    bn = 2000

    def body(a2_ref, a1_ref, r2_ref, w_ref, b_ref, oc_ref, om_ref):
        s2 = a2_ref[0] + a2_ref[1]
        cs = a1_ref[0] + a1_ref[1]
        c = jnp.maximum(cs[:, H:H + 1], 1.0)
        h2 = jnp.maximum(s2 / c + r2_ref[...], 0.0)
        z = lax.dot_general(h2, w_ref[...], (((1,), (1,)), ((), ())),
                            preferred_element_type=jnp.float32) + b_ref[...]
        oc_ref[...] = z[:, :c_out]
        om_ref[...] = z[:, c_out:]

    nout = c_out + m_out
    return pl.pallas_call(
        body,
        grid=(n // bn,),
        in_specs=[pl.BlockSpec((NC, bn, H), lambda i: (0, i, 0)),
                  pl.BlockSpec((NC, bn, W1COL), lambda i: (0, i, 0)),
                  pl.BlockSpec((bn, H), lambda i: (i, 0)),
                  pl.BlockSpec((nout, H), lambda i: (0, 0)),
                  pl.BlockSpec((1, nout), lambda i: (0, 0))],
        out_specs=[pl.BlockSpec((bn, c_out), lambda i: (i, 0)),
                   pl.BlockSpec((bn, m_out), lambda i: (i, 0))],
        out_shape=[jax.ShapeDtypeStruct((n, c_out), jnp.float32),
                   jax.ShapeDtypeStruct((n, m_out), jnp.float32)],
    )(acc2, acc1, r2, wcat, bcat.reshape(1, nout))


def _make_edge_pass(width, nch, np_rows):
    rows_pt = np_rows // NS
    mesh = plsc.VectorSubcoreMesh(core_axis_name="c", subcore_axis_name="s")

    @functools.partial(
        pl.kernel, mesh=mesh,
        compiler_params=pltpu.CompilerParams(use_tc_tiling_on_sc=False),
        out_type=jax.ShapeDtypeStruct((NC, np_rows, width), jnp.float32),
        scratch_types=[
            pltpu.VMEM((nch, CHUNK), jnp.int32),
            pltpu.VMEM((nch, CHUNK), jnp.int32),
            [pltpu.VMEM((CHUNK, width), jnp.float32)] * 10,
            pltpu.VMEM((16, width), jnp.float32),
            pltpu.VMEM_SHARED((np_rows, width), jnp.float32),
            pltpu.VMEM_SHARED((np_rows, width), jnp.float32),
            [pltpu.SemaphoreType.DMA] * 10,
            [pltpu.SemaphoreType.DMA] * 10,
        ],
    )
    def edge_pass(ytab, src_idx, dst_idx, out_acc,
                  src_v, dst_v, rows, zbuf, acc_sh, tab_sh, sg, ss):
        cid = lax.axis_index("c")
        sid = lax.axis_index("s")
        wid = sid * NC + cid

        for co in (0, 16, width - 16):
            for r in range(16):
                zbuf[r, pl.ds(co, 16)] = jnp.zeros((16,), jnp.float32)
        row0 = sid * rows_pt

        def zloop(k, carry):
            pltpu.sync_copy(zbuf, acc_sh.at[pl.ds(row0 + k * 16, 16)])
            return carry
        lax.fori_loop(0, rows_pt // 16, zloop, 0)

        pltpu.sync_copy(src_idx.at[wid], src_v)
        pltpu.sync_copy(dst_idx.at[wid], dst_v)
        pltpu.sync_copy(ytab.at[pl.ds(row0, rows_pt)],
                        tab_sh.at[pl.ds(row0, rows_pt)])
        plsc.subcore_barrier()

        def g_issue(j, b):
            pltpu.async_copy(tab_sh.at[src_v.at[j]], rows[b], sg[b])

        def g_wait(b):
            pltpu.make_async_copy(tab_sh.at[src_v.at[0]], rows[b], sg[b]).wait()

        def s_issue(j, b):
            pltpu.async_copy(rows[b], acc_sh.at[dst_v.at[j]], ss[b], add=True)

        def s_wait(b):
            pltpu.make_async_copy(rows[b], acc_sh.at[dst_v.at[0]], ss[b]).wait()

        nb, look = 10, 5
        for b in range(look):
            g_issue(b, b)
        for b in range(nb):
            if b >= look:
                s_wait(b - look)
            g_issue(b + look, (b + look) % nb)
            g_wait(b)
            s_issue(b, b)

        def body(g, carry):
            for b in range(nb):
                j = nb * g + b
                s_wait((b + look) % nb)
                g_issue(jnp.minimum(j + look, nch - 1), (b + look) % nb)
                g_wait(b)
                s_issue(j, b)
            return carry
        lax.fori_loop(1, nch // nb, body, 0)

        for b in range(look):
            g_wait(b)
        for b in range(nb - look, nb):
            s_wait(b)

        plsc.subcore_barrier()
        pltpu.sync_copy(acc_sh.at[pl.ds(row0, rows_pt)],
                        out_acc.at[cid, pl.ds(row0, rows_pt)])

    return edge_pass


def kernel(x, edge_index, Wl1, bl1, Wr1, Wl2, bl2, Wr2, Wh, bh, Wm, bm):
    n, d = x.shape
    e = edge_index.shape[1]

    epw = -(-e // (NW * 10 * CHUNK)) * 10 * CHUNK
    e_pad = NW * epw
    nch = epw // CHUNK
    np_rows = -(-(n + 1) // BN) * BN
    src = jnp.concatenate([edge_index[0], jnp.zeros((e_pad - e,), jnp.int32)])
    dst = jnp.concatenate([edge_index[1], jnp.full((e_pad - e,), n, jnp.int32)])
    src_r = src.reshape(NW, nch, CHUNK)
    dst_r = dst.reshape(NW, nch, CHUNK)
    w1 = jnp.concatenate([Wl1, Wr1], axis=0)
    w2 = jnp.concatenate([Wl2, Wr2], axis=0)
    wcat = jnp.concatenate([Wh, Wm], axis=0)
    bcat = jnp.concatenate([bh, bm])

    ytab, r1 = _tc_proj1(x, w1, bl1, n, np_rows)
    acc1 = _make_edge_pass(W1COL, nch, np_rows)(ytab, src_r, dst_r)
    y2, r2 = _tc_mid(acc1, r1, w2, bl2, np_rows)
    acc2 = _make_edge_pass(H, nch, np_rows)(y2, src_r, dst_r)
    return _tc_head(acc2, acc1, r2, wcat, bcat, n, Wh.shape[0], Wm.shape[0])
